# Initial kernel scaffold; baseline (speedup 1.0000x reference)
import functools

import jax
import jax.numpy as jnp
from jax import lax
from jax.experimental import pallas as pl
from jax.experimental.pallas import tpu as pltpu
from jax.experimental.pallas import tpu_sc as plsc

B = 16384
D = 64
G = 8
NUM_GENRES = 32

NC = 2
NS = 16
NW = NC * NS
BPW = B // NW
CHUNK = 128
KCH = BPW // CHUNK

BLK = 1024


def _sc_gather(title_table, idx3):
    mesh = plsc.VectorSubcoreMesh(
        core_axis_name="c", subcore_axis_name="s",
        num_cores=NC, num_subcores=NS)

    @functools.partial(
        pl.kernel,
        out_type=jax.ShapeDtypeStruct((B, D), jnp.float32),
        mesh=mesh,
        scratch_types=[
            pltpu.VMEM((KCH, CHUNK), jnp.int32),
            pltpu.VMEM((BPW, D), jnp.float32),
            pltpu.SemaphoreType.DMA,
        ],
    )
    def k(table_hbm, idx_hbm, out_hbm, idx_v, rows_v, sem):
        wid = lax.axis_index("s") * NC + lax.axis_index("c")
        pltpu.sync_copy(idx_hbm.at[wid], idx_v)
        cps = []
        for j in range(KCH):
            cps.append(pltpu.async_copy(
                table_hbm.at[idx_v.at[j]],
                rows_v.at[pl.ds(j * CHUNK, CHUNK)],
                sem))
        for cp in cps:
            cp.wait()
        pltpu.sync_copy(rows_v, out_hbm.at[pl.ds(wid * BPW, BPW)])

    return k(title_table, idx3)


def _tc_body(title_ref, genres_ref, gt_ref, w_ref, b_ref, out_ref):
    g = genres_ref[...]
    cls = lax.broadcasted_iota(jnp.int32, (1, NUM_GENRES), 1)
    counts = jnp.zeros((BLK, NUM_GENRES), jnp.float32)
    for j in range(G):
        counts += (g[:, j:j + 1] == cls).astype(jnp.float32)
    genre_emb = jnp.dot(counts, gt_ref[...],
                        preferred_element_type=jnp.float32)
    comb = jnp.concatenate([title_ref[...], genre_emb], axis=1)
    out = jnp.dot(comb, w_ref[...],
                  preferred_element_type=jnp.float32) + b_ref[...]
    out_ref[...] = jnp.maximum(out, 0.0)


def _tc_combine(title_g, movie_genres, genre_table, W, b2):
    return pl.pallas_call(
        _tc_body,
        out_shape=jax.ShapeDtypeStruct((B, D), jnp.float32),
        grid=(B // BLK,),
        in_specs=[
            pl.BlockSpec((BLK, D), lambda i: (i, 0)),
            pl.BlockSpec((BLK, G), lambda i: (i, 0)),
            pl.BlockSpec((NUM_GENRES, D), lambda i: (0, 0)),
            pl.BlockSpec((2 * D, D), lambda i: (0, 0)),
            pl.BlockSpec((1, D), lambda i: (0, 0)),
        ],
        out_specs=pl.BlockSpec((BLK, D), lambda i: (i, 0)),
    )(title_g, movie_genres, genre_table, W, b2)


def kernel(movie_title, movie_genres, title_table, genre_table, W, b):
    idx3 = movie_title.reshape(NW, KCH, CHUNK)
    title_g = _sc_gather(title_table, idx3)
    return _tc_combine(title_g, movie_genres, genre_table, W,
                       b.reshape(1, D))


# speedup vs baseline: 1.2496x; 1.2496x over previous
import functools

import jax
import jax.numpy as jnp
from jax import lax
from jax.experimental import pallas as pl
from jax.experimental.pallas import tpu as pltpu
from jax.experimental.pallas import tpu_sc as plsc

B = 16384
D = 64
G = 8
NUM_GENRES = 32

NC = 2
NS = 16
NW = NC * NS
BPW = B // NW
CHUNK = 128
KCH = BPW // CHUNK

BLK = 1024


def _sc_gather(title_table, idx3):
    mesh = plsc.VectorSubcoreMesh(
        core_axis_name="c", subcore_axis_name="s",
        num_cores=NC, num_subcores=NS)

    @functools.partial(
        pl.kernel,
        out_type=jax.ShapeDtypeStruct((B, D), jnp.float32),
        mesh=mesh,
        scratch_types=[
            pltpu.VMEM((KCH, CHUNK), jnp.int32),
            pltpu.VMEM((BPW, D), jnp.float32),
            pltpu.SemaphoreType.DMA,
        ],
        compiler_params=pltpu.CompilerParams(use_tc_tiling_on_sc=False),
    )
    def k(table_hbm, idx_hbm, out_hbm, idx_v, rows_v, sem):
        wid = lax.axis_index("s") * NC + lax.axis_index("c")
        pltpu.sync_copy(idx_hbm.at[wid], idx_v)
        cps = []
        for j in range(KCH):
            cps.append(pltpu.async_copy(
                table_hbm.at[idx_v.at[j]],
                rows_v.at[pl.ds(j * CHUNK, CHUNK)],
                sem))
        for cp in cps:
            cp.wait()
        pltpu.sync_copy(rows_v, out_hbm.at[pl.ds(wid * BPW, BPW)])

    return k(title_table, idx3)


def _tc_body(title_ref, genres_ref, gt_ref, w_ref, b_ref, out_ref):
    g = genres_ref[...]
    cls = lax.broadcasted_iota(jnp.int32, (1, NUM_GENRES), 1)
    counts = jnp.zeros((BLK, NUM_GENRES), jnp.float32)
    for j in range(G):
        counts += (g[:, j:j + 1] == cls).astype(jnp.float32)
    genre_emb = jnp.dot(counts, gt_ref[...],
                        preferred_element_type=jnp.float32)
    comb = jnp.concatenate([title_ref[...], genre_emb], axis=1)
    out = jnp.dot(comb, w_ref[...],
                  preferred_element_type=jnp.float32) + b_ref[...]
    out_ref[...] = jnp.maximum(out, 0.0)


def _tc_combine(title_g, movie_genres, genre_table, W, b2):
    return pl.pallas_call(
        _tc_body,
        out_shape=jax.ShapeDtypeStruct((B, D), jnp.float32),
        grid=(B // BLK,),
        in_specs=[
            pl.BlockSpec((BLK, D), lambda i: (i, 0)),
            pl.BlockSpec((BLK, G), lambda i: (i, 0)),
            pl.BlockSpec((NUM_GENRES, D), lambda i: (0, 0)),
            pl.BlockSpec((2 * D, D), lambda i: (0, 0)),
            pl.BlockSpec((1, D), lambda i: (0, 0)),
        ],
        out_specs=pl.BlockSpec((BLK, D), lambda i: (i, 0)),
    )(title_g, movie_genres, genre_table, W, b2)


def kernel(movie_title, movie_genres, title_table, genre_table, W, b):
    idx3 = movie_title.reshape(NW, KCH, CHUNK)
    title_g = _sc_gather(title_table, idx3)
    return _tc_combine(title_g, movie_genres, genre_table, W,
                       b.reshape(1, D))
